# baseline (speedup 1.0000x reference)
import functools

import jax
import jax.numpy as jnp
from jax import lax
from jax.experimental import pallas as pl
from jax.experimental.pallas import tpu as pltpu
from jax.experimental.pallas import tpu_sc as plsc

_D = 2048
_E = 8
_T = 2048
_RB = 128
_NB = (_T * 2 + _E * _RB) // _RB
_PAD_T = _NB * _RB
_CHUNK = 128


def _router_body(x_ref, rw_ref, rb_ref, pos0_ref, pos1_ref, w0_ref, w1_ref,
                 be_ref, slot_ref, nxe_ref, nxe2_ref):
    logits = jnp.dot(x_ref[...], rw_ref[...], preferred_element_type=jnp.float32)
    logits = logits + rb_ref[...]
    m = jnp.max(logits, axis=1, keepdims=True)
    p = jnp.exp(logits - m)
    eio = lax.broadcasted_iota(jnp.int32, p.shape, 1)
    m1 = jnp.max(p, axis=1, keepdims=True)
    i1 = jnp.min(jnp.where(p == m1, eio, _E), axis=1, keepdims=True)
    sel1 = eio == i1
    pm = jnp.where(sel1, -1.0, p)
    m2 = jnp.max(pm, axis=1, keepdims=True)
    i2 = jnp.min(jnp.where(pm == m2, eio, _E), axis=1, keepdims=True)
    sel2 = eio == i2
    s = m1 + m2
    w0_ref[...] = m1 / s
    w1_ref[...] = m2 / s

    a0 = sel1.astype(jnp.float32)
    a1 = sel2.astype(jnp.float32)
    counts = jnp.sum(a0 + a1, axis=0, keepdims=True)
    pc = jnp.ceil(counts / _RB) * _RB
    ei = lax.broadcasted_iota(jnp.int32, (_E, _E), 0)
    ej = lax.broadcasted_iota(jnp.int32, (_E, _E), 1)
    upper = (ei < ej).astype(jnp.float32)
    off = jnp.dot(pc, upper, preferred_element_type=jnp.float32)
    total = jnp.sum(pc, axis=1, keepdims=True)

    bio = lax.broadcasted_iota(jnp.int32, (1, 128), 1).astype(jnp.float32)
    row_start = bio * _RB
    acc = jnp.zeros((1, 128), jnp.float32)
    for e in range(_E):
        off_e = lax.slice(off, (0, e), (1, e + 1))
        acc = acc + (row_start >= off_e).astype(jnp.float32)
    be = jnp.where(row_start < total, acc - 1.0, -1.0)
    be_ref[...] = be.astype(jnp.int32)

    eio8 = lax.broadcasted_iota(jnp.int32, (1, _E), 1)
    nz = (pc > 0.0).astype(jnp.float32)
    incl = jnp.dot(nz, (ei <= ej).astype(jnp.float32),
                   preferred_element_type=jnp.float32)
    rix = incl - 1.0
    nxte = jnp.full((1, _E), -1.0)
    for ep in range(_E - 1, -1, -1):
        nz_ep = lax.slice(nz, (0, ep), (1, ep + 1))
        nxte = jnp.where((eio8 < ep) & (nz_ep > 0.0), float(ep), nxte)
    nxte2 = jnp.full((1, _E), -1.0)
    for e in range(_E):
        nxte_e = lax.slice(nxte, (0, e), (1, e + 1))
        nxte2 = jnp.where(nxte == float(e), nxte_e, nxte2)
    slotv = jnp.zeros((1, 128), jnp.float32)
    nxev = jnp.full((1, 128), -1.0)
    nxe2v = jnp.full((1, 128), -1.0)
    for e in range(_E):
        rix_e = lax.slice(rix, (0, e), (1, e + 1))
        nxte_e = lax.slice(nxte, (0, e), (1, e + 1))
        nxte2_e = lax.slice(nxte2, (0, e), (1, e + 1))
        iscur = be == float(e)
        slotv = jnp.where(iscur, rix_e, slotv)
        nxev = jnp.where(iscur, nxte_e, nxev)
        nxe2v = jnp.where(iscur, nxte2_e, nxe2v)
    slotv = slotv - 3.0 * jnp.floor(slotv / 3.0)
    slot_ref[...] = slotv.astype(jnp.int32)
    nxe_ref[...] = nxev.astype(jnp.int32)
    nxe2_ref[...] = nxe2v.astype(jnp.int32)

    ci = lax.broadcasted_iota(jnp.int32, (_CHUNK, _CHUNK), 0)
    cj = lax.broadcasted_iota(jnp.int32, (_CHUNK, _CHUNK), 1)
    tri = (cj < ci).astype(jnp.float32)
    carry = jnp.zeros((1, _E), jnp.float32)
    for k, (a, pos_ref) in enumerate(((a0, pos0_ref), (a1, pos1_ref))):
        for c in range(_T // _CHUNK):
            sl = lax.slice(a, (c * _CHUNK, 0), ((c + 1) * _CHUNK, _E))
            cum = jnp.dot(tri, sl, preferred_element_type=jnp.float32) + carry
            posc = jnp.sum(sl * (cum + off), axis=1, keepdims=True)
            pos_ref[c * _CHUNK:(c + 1) * _CHUNK, :] = posc.astype(jnp.int32)
            carry = carry + jnp.sum(sl, axis=0, keepdims=True)


def _run_router(xt, router_W, router_b):
    return pl.pallas_call(
        _router_body,
        out_shape=[
            jax.ShapeDtypeStruct((_T, 1), jnp.int32),
            jax.ShapeDtypeStruct((_T, 1), jnp.int32),
            jax.ShapeDtypeStruct((_T, 1), jnp.float32),
            jax.ShapeDtypeStruct((_T, 1), jnp.float32),
            jax.ShapeDtypeStruct((1, 128), jnp.int32),
            jax.ShapeDtypeStruct((1, 128), jnp.int32),
            jax.ShapeDtypeStruct((1, 128), jnp.int32),
            jax.ShapeDtypeStruct((1, 128), jnp.int32),
        ],
    )(xt, router_W, router_b.reshape(1, _E))


def _make_dispatch():
    mesh = plsc.VectorSubcoreMesh(core_axis_name="c", subcore_axis_name="s")
    info = plsc.get_sparse_core_info()
    nw = info.num_cores * info.num_subcores
    tpw = _T // nw
    ch = 32

    @functools.partial(
        pl.kernel, mesh=mesh,
        out_type=jax.ShapeDtypeStruct((_PAD_T, _D), jnp.float32),
        scratch_types=[
            pltpu.VMEM((ch, _D), jnp.float32),
            pltpu.VMEM((ch,), jnp.int32),
            pltpu.VMEM((ch,), jnp.int32),
            pltpu.SemaphoreType.DMA,
            pltpu.SemaphoreType.DMA,
        ],
    )
    def dispatch(x_hbm, pos0_hbm, pos1_hbm, xs_hbm,
                 xbuf, idx0, idx1, sem0, sem1):
        wid = lax.axis_index("s") * info.num_cores + lax.axis_index("c")
        base = wid * tpw
        for c in range(tpw // ch):
            b0 = base + c * ch
            pltpu.sync_copy(x_hbm.at[pl.ds(b0, ch)], xbuf)
            pltpu.sync_copy(pos0_hbm.at[pl.ds(b0, ch)], idx0)
            pltpu.sync_copy(pos1_hbm.at[pl.ds(b0, ch)], idx1)
            cp0 = pltpu.async_copy(xbuf, xs_hbm.at[idx0], sem0)
            cp1 = pltpu.async_copy(xbuf, xs_hbm.at[idx1], sem1)
            cp0.wait()
            cp1.wait()

    return dispatch


def _gemm_body(be_ref, sl_ref, nx_ref, nx2_ref, xs_ref, w_hbm, b_ref, ys_ref,
               wbuf, sems):
    b = pl.program_id(0)
    cur = be_ref[b]
    slot = sl_ref[b]
    nxe = nx_ref[b]
    nxe2 = nx2_ref[b]
    prev = be_ref[jnp.maximum(b - 1, 0)]
    changed = jnp.logical_or(b == 0, cur != prev)

    @pl.when(b == 0)
    def _():
        pltpu.make_async_copy(w_hbm.at[cur], wbuf.at[slot],
                              sems.at[slot]).start()

        @pl.when(nxe >= 0)
        def _():
            nslot = jnp.where(slot == 2, 0, slot + 1)
            pltpu.make_async_copy(w_hbm.at[nxe], wbuf.at[nslot],
                                  sems.at[nslot]).start()

    @pl.when(jnp.logical_and(changed, cur >= 0))
    def _():
        pltpu.make_async_copy(w_hbm.at[cur], wbuf.at[slot],
                              sems.at[slot]).wait()

        @pl.when(nxe2 >= 0)
        def _():
            pslot = jnp.where(slot == 0, 2, slot - 1)
            pltpu.make_async_copy(w_hbm.at[nxe2], wbuf.at[pslot],
                                  sems.at[pslot]).start()

    @pl.when(cur >= 0)
    def _():
        acc = jnp.dot(xs_ref[...].astype(jnp.bfloat16),
                      wbuf[slot].astype(jnp.bfloat16),
                      preferred_element_type=jnp.float32)
        ys_ref[...] = acc + b_ref[0]


def _run_gemm(be, slot, nxe, nxe2, xs, expert_W, expert_b):
    def bmap(b, *_refs):
        be_ref = _refs[0]
        return (jnp.maximum(be_ref[b], 0), 0, 0)

    return pl.pallas_call(
        _gemm_body,
        grid_spec=pltpu.PrefetchScalarGridSpec(
            num_scalar_prefetch=4,
            grid=(_NB,),
            in_specs=[
                pl.BlockSpec((_RB, _D), lambda b, *_: (b, 0)),
                pl.BlockSpec(memory_space=pl.ANY),
                pl.BlockSpec((1, 1, _D), bmap),
            ],
            out_specs=pl.BlockSpec((_RB, _D), lambda b, *_: (b, 0)),
            scratch_shapes=[
                pltpu.VMEM((3, _D, _D), jnp.float32),
                pltpu.SemaphoreType.DMA((3,)),
            ],
        ),
        out_shape=jax.ShapeDtypeStruct((_PAD_T, _D), jnp.float32),
    )(be, slot, nxe, nxe2, xs, expert_W, expert_b.reshape(_E, 1, _D))


def _make_combine():
    mesh = plsc.VectorSubcoreMesh(core_axis_name="c", subcore_axis_name="s")
    info = plsc.get_sparse_core_info()
    nw = info.num_cores * info.num_subcores
    tpw = _T // nw
    ch = 8
    nch = tpw // ch

    @functools.partial(
        pl.kernel, mesh=mesh,
        out_type=jax.ShapeDtypeStruct((_T, _D), jnp.float32),
        scratch_types=[
            pltpu.VMEM((2 * ch, _D), jnp.float32),
            pltpu.VMEM((2 * ch, _D), jnp.float32),
            pltpu.VMEM((2 * ch,), jnp.int32),
            pltpu.VMEM((2 * ch,), jnp.int32),
            pltpu.VMEM((2 * ch,), jnp.float32),
            pltpu.VMEM((2 * ch,), jnp.float32),
            pltpu.SemaphoreType.DMA,
            pltpu.SemaphoreType.DMA,
        ],
    )
    def combine(ys_hbm, pos0_hbm, pos1_hbm, w0_hbm, w1_hbm, out_hbm,
                buf0, buf1, idx0, idx1, wcb0, wcb1, sem0, sem1):
        wid = lax.axis_index("s") * info.num_cores + lax.axis_index("c")
        base = wid * tpw
        bufs = (buf0, buf1)
        idxs = (idx0, idx1)
        wcbs = (wcb0, wcb1)
        sems = (sem0, sem1)

        def prep(c, par):
            b0 = base + c * ch
            pltpu.sync_copy(pos0_hbm.at[pl.ds(b0, ch)],
                            idxs[par].at[pl.ds(0, ch)])
            pltpu.sync_copy(pos1_hbm.at[pl.ds(b0, ch)],
                            idxs[par].at[pl.ds(ch, ch)])
            pltpu.sync_copy(w0_hbm.at[pl.ds(b0, ch)],
                            wcbs[par].at[pl.ds(0, ch)])
            pltpu.sync_copy(w1_hbm.at[pl.ds(b0, ch)],
                            wcbs[par].at[pl.ds(ch, ch)])
            return pltpu.async_copy(ys_hbm.at[idxs[par]], bufs[par], sems[par])

        dn = lax.GatherDimensionNumbers(
            offset_dims=(), collapsed_slice_dims=(0,), start_index_map=(0,))
        handles = [prep(0, 0), None]
        for c in range(nch):
            par = c % 2
            if c + 1 < nch:
                handles[1 - par] = prep(c + 1, 1 - par)
            handles[par].wait()
            buf = bufs[par]
            wv = wcbs[par][...]
            for r in range(ch):
                w0v = lax.gather(wv, jnp.full((16, 1), r, jnp.int32), dn,
                                 slice_sizes=(1,),
                                 mode=lax.GatherScatterMode.PROMISE_IN_BOUNDS)
                w1v = lax.gather(wv, jnp.full((16, 1), ch + r, jnp.int32), dn,
                                 slice_sizes=(1,),
                                 mode=lax.GatherScatterMode.PROMISE_IN_BOUNDS)

                def body(i, _):
                    for u in range(8):
                        slu = pl.ds(i * 128 + u * 16, 16)
                        buf[r, slu] = buf[r, slu] * w0v + buf[ch + r, slu] * w1v
                    return 0

                lax.fori_loop(0, _D // 128, body, 0)
            pltpu.sync_copy(buf.at[pl.ds(0, ch)],
                            out_hbm.at[pl.ds(base + c * ch, ch)])

    return combine


def kernel(x, router_W, router_b, expert_W, expert_b):
    B, S, D = x.shape
    xt = x.reshape(_T, _D)

    pos0, pos1, w0, w1, be_pad, slot_pad, nxe_pad, nxe2_pad = _run_router(
        xt, router_W, router_b)
    be = be_pad[0, :_NB]
    slot = slot_pad[0, :_NB]
    nxe = nxe_pad[0, :_NB]
    nxe2 = nxe2_pad[0, :_NB]
    p0 = pos0.reshape(_T)
    p1 = pos1.reshape(_T)

    xs = _make_dispatch()(xt, p0, p1)
    ys = _run_gemm(be, slot, nxe, nxe2, xs, expert_W, expert_b)
    out = _make_combine()(ys, p0, p1, w0.reshape(_T), w1.reshape(_T))
    return out.reshape(B, S, D)


# speedup vs baseline: 1.0374x; 1.0374x over previous
import functools

import jax
import jax.numpy as jnp
from jax import lax
from jax.experimental import pallas as pl
from jax.experimental.pallas import tpu as pltpu
from jax.experimental.pallas import tpu_sc as plsc

_D = 2048
_E = 8
_T = 2048
_RB = 256
_NB = (_T * 2 + _E * _RB) // _RB
_PAD_T = _NB * _RB
_CHUNK = 128


def _router_body(x_ref, rw_ref, rb_ref, pos0_ref, pos1_ref, w0_ref, w1_ref,
                 be_ref, slot_ref, nxe_ref, nxe2_ref):
    logits = jnp.dot(x_ref[...], rw_ref[...], preferred_element_type=jnp.float32)
    logits = logits + rb_ref[...]
    m = jnp.max(logits, axis=1, keepdims=True)
    p = jnp.exp(logits - m)
    eio = lax.broadcasted_iota(jnp.int32, p.shape, 1)
    m1 = jnp.max(p, axis=1, keepdims=True)
    i1 = jnp.min(jnp.where(p == m1, eio, _E), axis=1, keepdims=True)
    sel1 = eio == i1
    pm = jnp.where(sel1, -1.0, p)
    m2 = jnp.max(pm, axis=1, keepdims=True)
    i2 = jnp.min(jnp.where(pm == m2, eio, _E), axis=1, keepdims=True)
    sel2 = eio == i2
    s = m1 + m2
    w0_ref[...] = m1 / s
    w1_ref[...] = m2 / s

    a0 = sel1.astype(jnp.float32)
    a1 = sel2.astype(jnp.float32)
    counts = jnp.sum(a0 + a1, axis=0, keepdims=True)
    pc = jnp.ceil(counts / _RB) * _RB
    ei = lax.broadcasted_iota(jnp.int32, (_E, _E), 0)
    ej = lax.broadcasted_iota(jnp.int32, (_E, _E), 1)
    upper = (ei < ej).astype(jnp.float32)
    off = jnp.dot(pc, upper, preferred_element_type=jnp.float32)
    total = jnp.sum(pc, axis=1, keepdims=True)

    bio = lax.broadcasted_iota(jnp.int32, (1, 128), 1).astype(jnp.float32)
    row_start = bio * _RB
    acc = jnp.zeros((1, 128), jnp.float32)
    for e in range(_E):
        off_e = lax.slice(off, (0, e), (1, e + 1))
        acc = acc + (row_start >= off_e).astype(jnp.float32)
    be = jnp.where(row_start < total, acc - 1.0, -1.0)
    be_ref[...] = be.astype(jnp.int32)

    eio8 = lax.broadcasted_iota(jnp.int32, (1, _E), 1)
    nz = (pc > 0.0).astype(jnp.float32)
    incl = jnp.dot(nz, (ei <= ej).astype(jnp.float32),
                   preferred_element_type=jnp.float32)
    rix = incl - 1.0
    nxte = jnp.full((1, _E), -1.0)
    for ep in range(_E - 1, -1, -1):
        nz_ep = lax.slice(nz, (0, ep), (1, ep + 1))
        nxte = jnp.where((eio8 < ep) & (nz_ep > 0.0), float(ep), nxte)
    nxte2 = jnp.full((1, _E), -1.0)
    for e in range(_E):
        nxte_e = lax.slice(nxte, (0, e), (1, e + 1))
        nxte2 = jnp.where(nxte == float(e), nxte_e, nxte2)
    slotv = jnp.zeros((1, 128), jnp.float32)
    nxev = jnp.full((1, 128), -1.0)
    nxe2v = jnp.full((1, 128), -1.0)
    for e in range(_E):
        rix_e = lax.slice(rix, (0, e), (1, e + 1))
        nxte_e = lax.slice(nxte, (0, e), (1, e + 1))
        nxte2_e = lax.slice(nxte2, (0, e), (1, e + 1))
        iscur = be == float(e)
        slotv = jnp.where(iscur, rix_e, slotv)
        nxev = jnp.where(iscur, nxte_e, nxev)
        nxe2v = jnp.where(iscur, nxte2_e, nxe2v)
    slotv = slotv - 3.0 * jnp.floor(slotv / 3.0)
    slot_ref[...] = slotv.astype(jnp.int32)
    nxe_ref[...] = nxev.astype(jnp.int32)
    nxe2_ref[...] = nxe2v.astype(jnp.int32)

    ci = lax.broadcasted_iota(jnp.int32, (_CHUNK, _CHUNK), 0)
    cj = lax.broadcasted_iota(jnp.int32, (_CHUNK, _CHUNK), 1)
    tri = (cj < ci).astype(jnp.float32)
    carry = jnp.zeros((1, _E), jnp.float32)
    for k, (a, pos_ref) in enumerate(((a0, pos0_ref), (a1, pos1_ref))):
        for c in range(_T // _CHUNK):
            sl = lax.slice(a, (c * _CHUNK, 0), ((c + 1) * _CHUNK, _E))
            cum = jnp.dot(tri, sl, preferred_element_type=jnp.float32) + carry
            posc = jnp.sum(sl * (cum + off), axis=1, keepdims=True)
            pos_ref[c * _CHUNK:(c + 1) * _CHUNK, :] = posc.astype(jnp.int32)
            carry = carry + jnp.sum(sl, axis=0, keepdims=True)


def _run_router(xt, router_W, router_b):
    return pl.pallas_call(
        _router_body,
        out_shape=[
            jax.ShapeDtypeStruct((_T, 1), jnp.int32),
            jax.ShapeDtypeStruct((_T, 1), jnp.int32),
            jax.ShapeDtypeStruct((_T, 1), jnp.float32),
            jax.ShapeDtypeStruct((_T, 1), jnp.float32),
            jax.ShapeDtypeStruct((1, 128), jnp.int32),
            jax.ShapeDtypeStruct((1, 128), jnp.int32),
            jax.ShapeDtypeStruct((1, 128), jnp.int32),
            jax.ShapeDtypeStruct((1, 128), jnp.int32),
        ],
    )(xt, router_W, router_b.reshape(1, _E))


def _make_dispatch():
    mesh = plsc.VectorSubcoreMesh(core_axis_name="c", subcore_axis_name="s")
    info = plsc.get_sparse_core_info()
    nw = info.num_cores * info.num_subcores
    tpw = _T // nw
    ch = 32

    @functools.partial(
        pl.kernel, mesh=mesh,
        out_type=jax.ShapeDtypeStruct((_PAD_T, _D), jnp.float32),
        scratch_types=[
            pltpu.VMEM((ch, _D), jnp.float32),
            pltpu.VMEM((ch,), jnp.int32),
            pltpu.VMEM((ch,), jnp.int32),
            pltpu.SemaphoreType.DMA,
            pltpu.SemaphoreType.DMA,
        ],
    )
    def dispatch(x_hbm, pos0_hbm, pos1_hbm, xs_hbm,
                 xbuf, idx0, idx1, sem0, sem1):
        wid = lax.axis_index("s") * info.num_cores + lax.axis_index("c")
        base = wid * tpw
        for c in range(tpw // ch):
            b0 = base + c * ch
            pltpu.sync_copy(x_hbm.at[pl.ds(b0, ch)], xbuf)
            pltpu.sync_copy(pos0_hbm.at[pl.ds(b0, ch)], idx0)
            pltpu.sync_copy(pos1_hbm.at[pl.ds(b0, ch)], idx1)
            cp0 = pltpu.async_copy(xbuf, xs_hbm.at[idx0], sem0)
            cp1 = pltpu.async_copy(xbuf, xs_hbm.at[idx1], sem1)
            cp0.wait()
            cp1.wait()

    return dispatch


def _gemm_body(be_ref, sl_ref, nx_ref, nx2_ref, xs_ref, w_hbm, b_ref, ys_ref,
               wbuf, sems):
    b = pl.program_id(0)
    cur = be_ref[b]
    slot = sl_ref[b]
    nxe = nx_ref[b]
    nxe2 = nx2_ref[b]
    prev = be_ref[jnp.maximum(b - 1, 0)]
    changed = jnp.logical_or(b == 0, cur != prev)

    @pl.when(b == 0)
    def _():
        pltpu.make_async_copy(w_hbm.at[cur], wbuf.at[slot],
                              sems.at[slot]).start()

        @pl.when(nxe >= 0)
        def _():
            nslot = jnp.where(slot == 2, 0, slot + 1)
            pltpu.make_async_copy(w_hbm.at[nxe], wbuf.at[nslot],
                                  sems.at[nslot]).start()

    @pl.when(jnp.logical_and(changed, cur >= 0))
    def _():
        pltpu.make_async_copy(w_hbm.at[cur], wbuf.at[slot],
                              sems.at[slot]).wait()

        @pl.when(nxe2 >= 0)
        def _():
            pslot = jnp.where(slot == 0, 2, slot - 1)
            pltpu.make_async_copy(w_hbm.at[nxe2], wbuf.at[pslot],
                                  sems.at[pslot]).start()

    @pl.when(cur >= 0)
    def _():
        acc = jnp.dot(xs_ref[...].astype(jnp.bfloat16),
                      wbuf[slot].astype(jnp.bfloat16),
                      preferred_element_type=jnp.float32)
        ys_ref[...] = acc + b_ref[0]


def _run_gemm(be, slot, nxe, nxe2, xs, expert_W, expert_b):
    def bmap(b, *_refs):
        be_ref = _refs[0]
        return (jnp.maximum(be_ref[b], 0), 0, 0)

    return pl.pallas_call(
        _gemm_body,
        grid_spec=pltpu.PrefetchScalarGridSpec(
            num_scalar_prefetch=4,
            grid=(_NB,),
            in_specs=[
                pl.BlockSpec((_RB, _D), lambda b, *_: (b, 0)),
                pl.BlockSpec(memory_space=pl.ANY),
                pl.BlockSpec((1, 1, _D), bmap),
            ],
            out_specs=pl.BlockSpec((_RB, _D), lambda b, *_: (b, 0)),
            scratch_shapes=[
                pltpu.VMEM((3, _D, _D), jnp.float32),
                pltpu.SemaphoreType.DMA((3,)),
            ],
        ),
        out_shape=jax.ShapeDtypeStruct((_PAD_T, _D), jnp.float32),
    )(be, slot, nxe, nxe2, xs, expert_W, expert_b.reshape(_E, 1, _D))


def _make_combine():
    mesh = plsc.VectorSubcoreMesh(core_axis_name="c", subcore_axis_name="s")
    info = plsc.get_sparse_core_info()
    nw = info.num_cores * info.num_subcores
    tpw = _T // nw
    ch = 8
    nch = tpw // ch

    @functools.partial(
        pl.kernel, mesh=mesh,
        out_type=jax.ShapeDtypeStruct((_T, _D), jnp.float32),
        scratch_types=[
            pltpu.VMEM((2 * ch, _D), jnp.float32),
            pltpu.VMEM((2 * ch, _D), jnp.float32),
            pltpu.VMEM((2 * ch,), jnp.int32),
            pltpu.VMEM((2 * ch,), jnp.int32),
            pltpu.VMEM((2 * ch,), jnp.float32),
            pltpu.VMEM((2 * ch,), jnp.float32),
            pltpu.SemaphoreType.DMA,
            pltpu.SemaphoreType.DMA,
        ],
    )
    def combine(ys_hbm, pos0_hbm, pos1_hbm, w0_hbm, w1_hbm, out_hbm,
                buf0, buf1, idx0, idx1, wcb0, wcb1, sem0, sem1):
        wid = lax.axis_index("s") * info.num_cores + lax.axis_index("c")
        base = wid * tpw
        bufs = (buf0, buf1)
        idxs = (idx0, idx1)
        wcbs = (wcb0, wcb1)
        sems = (sem0, sem1)

        def prep(c, par):
            b0 = base + c * ch
            pltpu.sync_copy(pos0_hbm.at[pl.ds(b0, ch)],
                            idxs[par].at[pl.ds(0, ch)])
            pltpu.sync_copy(pos1_hbm.at[pl.ds(b0, ch)],
                            idxs[par].at[pl.ds(ch, ch)])
            pltpu.sync_copy(w0_hbm.at[pl.ds(b0, ch)],
                            wcbs[par].at[pl.ds(0, ch)])
            pltpu.sync_copy(w1_hbm.at[pl.ds(b0, ch)],
                            wcbs[par].at[pl.ds(ch, ch)])
            return pltpu.async_copy(ys_hbm.at[idxs[par]], bufs[par], sems[par])

        dn = lax.GatherDimensionNumbers(
            offset_dims=(), collapsed_slice_dims=(0,), start_index_map=(0,))
        handles = [prep(0, 0), None]
        for c in range(nch):
            par = c % 2
            if c + 1 < nch:
                handles[1 - par] = prep(c + 1, 1 - par)
            handles[par].wait()
            buf = bufs[par]
            wv = wcbs[par][...]
            for r in range(ch):
                w0v = lax.gather(wv, jnp.full((16, 1), r, jnp.int32), dn,
                                 slice_sizes=(1,),
                                 mode=lax.GatherScatterMode.PROMISE_IN_BOUNDS)
                w1v = lax.gather(wv, jnp.full((16, 1), ch + r, jnp.int32), dn,
                                 slice_sizes=(1,),
                                 mode=lax.GatherScatterMode.PROMISE_IN_BOUNDS)

                def body(i, _):
                    for u in range(8):
                        slu = pl.ds(i * 128 + u * 16, 16)
                        buf[r, slu] = buf[r, slu] * w0v + buf[ch + r, slu] * w1v
                    return 0

                lax.fori_loop(0, _D // 128, body, 0)
            pltpu.sync_copy(buf.at[pl.ds(0, ch)],
                            out_hbm.at[pl.ds(base + c * ch, ch)])

    return combine


def kernel(x, router_W, router_b, expert_W, expert_b):
    B, S, D = x.shape
    xt = x.reshape(_T, _D)

    pos0, pos1, w0, w1, be_pad, slot_pad, nxe_pad, nxe2_pad = _run_router(
        xt, router_W, router_b)
    be = be_pad.reshape(128)
    slot = slot_pad.reshape(128)
    nxe = nxe_pad.reshape(128)
    nxe2 = nxe2_pad.reshape(128)
    p0 = pos0.reshape(_T)
    p1 = pos1.reshape(_T)

    xs = _make_dispatch()(xt, p0, p1)
    ys = _run_gemm(be, slot, nxe, nxe2, xs, expert_W, expert_b)
    out = _make_combine()(ys, p0, p1, w0.reshape(_T), w1.reshape(_T))
    return out.reshape(B, S, D)
